# baseline (speedup 1.0000x reference)
import jax
import jax.numpy as jnp
from jax import lax
from jax.experimental import pallas as pl
from jax.experimental.pallas import tpu as pltpu
from jax.experimental.pallas import tpu_sc as plsc

_B = 16384
_D = 64
_NR = 1000
_NC = 2
_NS = 16
_NW = _NC * _NS
_BPW = _B // _NW
_HALF = _BPW // 2
_CH = 16
_NCH = _HALF // _CH


def _fire(ent_hbm, hvec, tvec, e1b, e2b, sems):
    for j in range(_CH):
        pltpu.async_copy(ent_hbm.at[pl.ds(hvec[j], 1), :], e1b.at[j],
                         sems[j % 4])
        pltpu.async_copy(ent_hbm.at[pl.ds(tvec[j], 1), :], e2b.at[j],
                         sems[(j + 2) % 4])


def _body(ent_hbm, rel_hbm, heads_hbm, rels_hbm, tails_hbm, out_hbm,
          hidx, ridx, tidx, rp, e1b2, e2b2, r_v, out_v,
          sem0, sem1, sem2, sem3, rsem):
    wid = lax.axis_index("s") * _NC + lax.axis_index("c")
    base = wid * _BPW

    pltpu.sync_copy(heads_hbm.at[pl.ds(base, _BPW)], hidx)
    pltpu.sync_copy(rels_hbm.at[pl.ds(base, _BPW)], ridx)
    pltpu.sync_copy(tails_hbm.at[pl.ds(base, _BPW)], tidx)

    for k in range(_BPW // 16):
        sl = pl.ds(k * 16, 16)
        rp[sl] = ridx[sl] >> 1

    lanes16 = lax.iota(jnp.int32, 16)
    ones16 = jnp.ones((16,), jnp.int32)
    bfly = [jnp.bitwise_xor(lanes16, sh) for sh in (8, 4, 2, 1)]
    dnums = lax.GatherDimensionNumbers(
        offset_dims=(), collapsed_slice_dims=(0,), start_index_map=(0,))

    def shuffle(v, idx):
        return lax.gather(v, idx[:, None], dnums, slice_sizes=(1,),
                          mode=lax.GatherScatterMode.PROMISE_IN_BOUNDS)

    def lanesum(v):
        for idx in bfly:
            v = v + shuffle(v, idx)
        return v

    def idx_chunk(half, k):
        sl = pl.ds(half * _HALF + k * _CH, _CH)
        return hidx[sl], tidx[sl]

    sems = (sem0, sem1, sem2, sem3)

    def drain(slot):
        dummy = ent_hbm.at[pl.ds(0, 1), :]
        for j in range(_CH):
            pltpu.make_async_copy(dummy, e1b2.at[slot, j], sems[j % 4]).wait()
            pltpu.make_async_copy(dummy, e2b2.at[slot, j],
                                  sems[(j + 2) % 4]).wait()

    def fire_chunk(half, k, slot):
        hv, tv = idx_chunk(half, k)
        _fire(ent_hbm, hv, tv, e1b2.at[slot], e2b2.at[slot], sems)

    def compute_chunk(half, k, slot):
        sl = pl.ds(half * _HALF + k * _CH, _CH)
        rparf = (ridx[sl] & ones16).astype(jnp.float32)
        s = jnp.zeros((16,), jnp.float32)
        for j in range(_CH):
            row = k * _CH + j
            jv = jnp.full((16,), j, jnp.int32)
            pr = shuffle(rparf, jv)
            acc = jnp.zeros((16,), jnp.float32)
            for c in range(_D // 16):
                lo = pl.ds(c * 16, 16)
                hi = pl.ds(64 + c * 16, 16)
                b1 = r_v[row, lo]
                b = b1 + pr * (r_v[row, hi] - b1)
                acc = acc + (e1b2[slot, j, 0, lo] * b) \
                    * e2b2[slot, j, 0, lo]
            s = jnp.where(lanes16 == j, lanesum(acc), s)
        out_v[pl.ds(k * _CH, _CH)] = 1.0 / (1.0 + jnp.exp(-s))

    for half in range(2):
        rcopies = []
        for q in range(_HALF // 128):
            isl = pl.ds(half * _HALF + q * 128, 128)
            vsl = pl.ds(q * 128, 128)
            rcopies.append(
                pltpu.async_copy(rel_hbm.at[rp.at[isl]], r_v.at[vsl], rsem))

        fire_chunk(half, 0, 0)

        def step(i, carry):
            a = i * 2
            fire_chunk(half, a + 1, 1)
            drain(0)
            compute_chunk(half, a, 0)
            fire_chunk(half, jnp.minimum(a + 2, _NCH - 1), 0)
            drain(1)
            compute_chunk(half, a + 1, 1)
            return carry

        for c in rcopies:
            c.wait()
        lax.fori_loop(0, _NCH // 2, step, 0)
        drain(0)
        pltpu.sync_copy(out_v, out_hbm.at[pl.ds(base + half * _HALF, _HALF)])


def kernel(entity_embedding, relation_embedding, heads, relations, tails):
    mesh = plsc.VectorSubcoreMesh(core_axis_name="c", subcore_axis_name="s")
    run = pl.kernel(
        _body,
        out_type=jax.ShapeDtypeStruct((_B,), jnp.float32),
        mesh=mesh,
        scratch_types=[
            pltpu.VMEM((_BPW,), jnp.int32),
            pltpu.VMEM((_BPW,), jnp.int32),
            pltpu.VMEM((_BPW,), jnp.int32),
            pltpu.VMEM((_BPW,), jnp.int32),
            pltpu.VMEM((2, _CH, 1, _D), jnp.float32),
            pltpu.VMEM((2, _CH, 1, _D), jnp.float32),
            pltpu.VMEM((_HALF, 2 * _D), jnp.float32),
            pltpu.VMEM((_HALF,), jnp.float32),
            pltpu.SemaphoreType.DMA,
            pltpu.SemaphoreType.DMA,
            pltpu.SemaphoreType.DMA,
            pltpu.SemaphoreType.DMA,
            pltpu.SemaphoreType.DMA,
        ],
    )
    return run(entity_embedding, relation_embedding.reshape(_NR // 2, 128),
               heads.astype(jnp.int32), relations.astype(jnp.int32),
               tails.astype(jnp.int32))


# speedup vs baseline: 1.0060x; 1.0060x over previous
import jax
import jax.numpy as jnp
from jax import lax
from jax.experimental import pallas as pl
from jax.experimental.pallas import tpu as pltpu
from jax.experimental.pallas import tpu_sc as plsc

_B = 16384
_D = 64
_NR = 1000
_NC = 2
_NS = 16
_NW = _NC * _NS
_BPW = _B // _NW
_HALF = _BPW // 2
_CH = 16
_NCH = _HALF // _CH


def _fire(ent_hbm, hvec, tvec, e1b, e2b, sem):
    for j in range(_CH):
        pltpu.async_copy(ent_hbm.at[pl.ds(hvec[j], 1), :], e1b.at[j], sem)
        pltpu.async_copy(ent_hbm.at[pl.ds(tvec[j], 1), :], e2b.at[j], sem)


def _body(ent_hbm, rel_hbm, heads_hbm, rels_hbm, tails_hbm, out_hbm,
          hidx, ridx, tidx, rp, e1b2, e2b2, r_v, out_v, sem, rsem):
    wid = lax.axis_index("s") * _NC + lax.axis_index("c")
    base = wid * _BPW

    pltpu.sync_copy(heads_hbm.at[pl.ds(base, _BPW)], hidx)
    pltpu.sync_copy(rels_hbm.at[pl.ds(base, _BPW)], ridx)
    pltpu.sync_copy(tails_hbm.at[pl.ds(base, _BPW)], tidx)

    for k in range(_BPW // 16):
        sl = pl.ds(k * 16, 16)
        rp[sl] = ridx[sl] >> 1

    lanes16 = lax.iota(jnp.int32, 16)
    ones16 = jnp.ones((16,), jnp.int32)
    bfly = [jnp.bitwise_xor(lanes16, sh) for sh in (8, 4, 2, 1)]
    dnums = lax.GatherDimensionNumbers(
        offset_dims=(), collapsed_slice_dims=(0,), start_index_map=(0,))

    def shuffle(v, idx):
        return lax.gather(v, idx[:, None], dnums, slice_sizes=(1,),
                          mode=lax.GatherScatterMode.PROMISE_IN_BOUNDS)

    def lanesum(v):
        for idx in bfly:
            v = v + shuffle(v, idx)
        return v

    def idx_chunk(half, k):
        sl = pl.ds(half * _HALF + k * _CH, _CH)
        return hidx[sl], tidx[sl]

    def drain(slot):
        dummy = ent_hbm.at[pl.ds(0, _CH), :]
        pltpu.make_async_copy(dummy, e1b2.at[slot, :, 0, :], sem).wait()
        pltpu.make_async_copy(dummy, e2b2.at[slot, :, 0, :], sem).wait()

    def fire_chunk(half, k, slot):
        hv, tv = idx_chunk(half, k)
        _fire(ent_hbm, hv, tv, e1b2.at[slot], e2b2.at[slot], sem)

    def compute_chunk(half, k, slot):
        sl = pl.ds(half * _HALF + k * _CH, _CH)
        rparf = (ridx[sl] & ones16).astype(jnp.float32)
        s = jnp.zeros((16,), jnp.float32)
        for j in range(_CH):
            row = k * _CH + j
            jv = jnp.full((16,), j, jnp.int32)
            pr = shuffle(rparf, jv)
            acc = jnp.zeros((16,), jnp.float32)
            for c in range(_D // 16):
                lo = pl.ds(c * 16, 16)
                hi = pl.ds(64 + c * 16, 16)
                b1 = r_v[row, lo]
                b = b1 + pr * (r_v[row, hi] - b1)
                acc = acc + (e1b2[slot, j, 0, lo] * b) \
                    * e2b2[slot, j, 0, lo]
            s = jnp.where(lanes16 == j, lanesum(acc), s)
        out_v[pl.ds(k * _CH, _CH)] = 1.0 / (1.0 + jnp.exp(-s))

    for half in range(2):
        rcopies = []
        for q in range(_HALF // 128):
            isl = pl.ds(half * _HALF + q * 128, 128)
            vsl = pl.ds(q * 128, 128)
            rcopies.append(
                pltpu.async_copy(rel_hbm.at[rp.at[isl]], r_v.at[vsl], rsem))

        fire_chunk(half, 0, 0)

        def step(i, carry):
            a = i * 2
            fire_chunk(half, a + 1, 1)
            drain(0)
            compute_chunk(half, a, 0)
            fire_chunk(half, jnp.minimum(a + 2, _NCH - 1), 0)
            drain(1)
            compute_chunk(half, a + 1, 1)
            return carry

        for c in rcopies:
            c.wait()
        lax.fori_loop(0, _NCH // 2, step, 0)
        drain(0)
        pltpu.sync_copy(out_v, out_hbm.at[pl.ds(base + half * _HALF, _HALF)])


def kernel(entity_embedding, relation_embedding, heads, relations, tails):
    mesh = plsc.VectorSubcoreMesh(core_axis_name="c", subcore_axis_name="s")
    run = pl.kernel(
        _body,
        out_type=jax.ShapeDtypeStruct((_B,), jnp.float32),
        mesh=mesh,
        scratch_types=[
            pltpu.VMEM((_BPW,), jnp.int32),
            pltpu.VMEM((_BPW,), jnp.int32),
            pltpu.VMEM((_BPW,), jnp.int32),
            pltpu.VMEM((_BPW,), jnp.int32),
            pltpu.VMEM((2, _CH, 1, _D), jnp.float32),
            pltpu.VMEM((2, _CH, 1, _D), jnp.float32),
            pltpu.VMEM((_HALF, 2 * _D), jnp.float32),
            pltpu.VMEM((_HALF,), jnp.float32),
            pltpu.SemaphoreType.DMA,
            pltpu.SemaphoreType.DMA,
        ],
    )
    return run(entity_embedding, relation_embedding.reshape(_NR // 2, 128),
               heads.astype(jnp.int32), relations.astype(jnp.int32),
               tails.astype(jnp.int32))
